# baseline (speedup 1.0000x reference)
import functools

import jax
import jax.numpy as jnp
from jax import lax
from jax.experimental import pallas as pl
from jax.experimental.pallas import tpu as pltpu
from jax.experimental.pallas import tpu_sc as plsc

N, E, D = 10000, 320000, 128
NC, NS = 2, 16
C = 50
K = E // (NC * NS * C)
NB = 4
assert E == NC * NS * K * C and K % NB == 0
NP = 10240
RPT = NP // NS
ZR = 8


def _sc_agg_body(x_hbm, src_hbm, dst_hbm, part_hbm,
                 sidx, didx, r0, r1, r2, r3, zbuf, agg_sh, s0, s1, s2, s3):
    cid = lax.axis_index("c")
    sid = lax.axis_index("s")
    rows = (r0, r1, r2, r3)
    sems = (s0, s1, s2, s3)

    pltpu.sync_copy(src_hbm.at[cid, sid], sidx)
    pltpu.sync_copy(dst_hbm.at[cid, sid], didx)

    for b in range(NB):
        pltpu.async_copy(x_hbm.at[sidx.at[b]], rows[b], sems[b])

    @pl.loop(0, ZR)
    def _zero_rows(r):
        @pl.loop(0, D // 16)
        def _zero_vecs(i):
            zbuf[r, pl.ds(i * 16, 16)] = jnp.zeros((16,), jnp.float32)

    base = sid * RPT

    @pl.loop(0, RPT // ZR)
    def _fill(z):
        pltpu.sync_copy(zbuf, agg_sh.at[pl.ds(base + z * ZR, ZR)])

    plsc.subcore_barrier()

    @pl.loop(0, K - NB, step=NB)
    def _edges(j):
        for b in range(NB):
            pltpu.make_async_copy(x_hbm.at[sidx.at[j + b]], rows[b], sems[b]).wait()
            pltpu.sync_copy(rows[b], agg_sh.at[didx.at[j + b]], add=True)
            pltpu.async_copy(x_hbm.at[sidx.at[j + NB + b]], rows[b], sems[b])

    for b in range(NB):
        pltpu.make_async_copy(x_hbm.at[sidx.at[K - NB + b]], rows[b], sems[b]).wait()
        pltpu.sync_copy(rows[b], agg_sh.at[didx.at[K - NB + b]], add=True)

    plsc.subcore_barrier()

    pltpu.sync_copy(agg_sh.at[pl.ds(base, RPT)],
                    part_hbm.at[cid, pl.ds(base, RPT)])


_sc_agg = functools.partial(
    pl.kernel,
    out_type=jax.ShapeDtypeStruct((NC, NP, D), jnp.float32),
    mesh=plsc.VectorSubcoreMesh(core_axis_name="c", subcore_axis_name="s"),
    scratch_types=[
        pltpu.VMEM((K, C), jnp.int32),
        pltpu.VMEM((K, C), jnp.int32),
        pltpu.VMEM((C, D), jnp.float32),
        pltpu.VMEM((C, D), jnp.float32),
        pltpu.VMEM((C, D), jnp.float32),
        pltpu.VMEM((C, D), jnp.float32),
        pltpu.VMEM((ZR, D), jnp.float32),
        pltpu.VMEM_SHARED((NP, D), jnp.float32),
        pltpu.SemaphoreType.DMA,
        pltpu.SemaphoreType.DMA,
        pltpu.SemaphoreType.DMA,
        pltpu.SemaphoreType.DMA,
    ],
    compiler_params=pltpu.CompilerParams(use_tc_tiling_on_sc=False),
)(_sc_agg_body)


BN = 1000


def _tc_body(p_ref, x_ref, wr_ref, wx_ref, b_ref, o_ref):
    agg = p_ref[0] + p_ref[1]
    dn = (((1,), (1,)), ((), ()))
    acc = lax.dot_general(agg, wr_ref[...], dn, preferred_element_type=jnp.float32)
    acc += lax.dot_general(x_ref[...], wx_ref[...], dn, preferred_element_type=jnp.float32)
    o_ref[...] = jnp.maximum(acc + b_ref[...], 0.0)


def kernel(x, edge_index, W_rel, b_rel, W_root):
    src = edge_index[0].reshape(NC, NS, K, C)
    dst = edge_index[1].reshape(NC, NS, K, C)
    part = _sc_agg(x, src, dst)

    out = pl.pallas_call(
        _tc_body,
        grid=(N // BN,),
        in_specs=[
            pl.BlockSpec((NC, BN, D), lambda i: (0, i, 0)),
            pl.BlockSpec((BN, D), lambda i: (i, 0)),
            pl.BlockSpec((D, D), lambda i: (0, 0)),
            pl.BlockSpec((D, D), lambda i: (0, 0)),
            pl.BlockSpec((1, D), lambda i: (0, 0)),
        ],
        out_specs=pl.BlockSpec((BN, D), lambda i: (i, 0)),
        out_shape=jax.ShapeDtypeStruct((N, D), jnp.float32),
    )(part, x, W_rel, W_root, b_rel.reshape(1, D))
    return out


# speedup vs baseline: 1.0055x; 1.0055x over previous
import functools

import jax
import jax.numpy as jnp
from jax import lax
from jax.experimental import pallas as pl
from jax.experimental.pallas import tpu as pltpu
from jax.experimental.pallas import tpu_sc as plsc

N, E, D = 10000, 320000, 128
NC, NS = 2, 16
C = 100
K = E // (NC * NS * C)
NB = 2
assert E == NC * NS * K * C and K % NB == 0
NP = 10240
RPT = NP // NS
ZR = 8


def _sc_agg_body(x_hbm, src_hbm, dst_hbm, part_hbm,
                 sidx, didx, r0, r1, zbuf, agg_sh, s0, s1):
    cid = lax.axis_index("c")
    sid = lax.axis_index("s")
    rows = (r0, r1)
    sems = (s0, s1)

    pltpu.sync_copy(src_hbm.at[cid, sid], sidx)
    pltpu.sync_copy(dst_hbm.at[cid, sid], didx)

    for b in range(NB):
        pltpu.async_copy(x_hbm.at[sidx.at[b]], rows[b], sems[b])

    @pl.loop(0, ZR)
    def _zero_rows(r):
        @pl.loop(0, D // 16)
        def _zero_vecs(i):
            zbuf[r, pl.ds(i * 16, 16)] = jnp.zeros((16,), jnp.float32)

    base = sid * RPT

    @pl.loop(0, RPT // ZR)
    def _fill(z):
        pltpu.sync_copy(zbuf, agg_sh.at[pl.ds(base + z * ZR, ZR)])

    plsc.subcore_barrier()

    @pl.loop(0, K - NB, step=NB)
    def _edges(j):
        for b in range(NB):
            pltpu.make_async_copy(x_hbm.at[sidx.at[j + b]], rows[b], sems[b]).wait()
            pltpu.sync_copy(rows[b], agg_sh.at[didx.at[j + b]], add=True)
            pltpu.async_copy(x_hbm.at[sidx.at[j + NB + b]], rows[b], sems[b])

    for b in range(NB):
        pltpu.make_async_copy(x_hbm.at[sidx.at[K - NB + b]], rows[b], sems[b]).wait()
        pltpu.sync_copy(rows[b], agg_sh.at[didx.at[K - NB + b]], add=True)

    plsc.subcore_barrier()

    pltpu.sync_copy(agg_sh.at[pl.ds(base, RPT)],
                    part_hbm.at[cid, pl.ds(base, RPT)])


_sc_agg = functools.partial(
    pl.kernel,
    out_type=jax.ShapeDtypeStruct((NC, NP, D), jnp.float32),
    mesh=plsc.VectorSubcoreMesh(core_axis_name="c", subcore_axis_name="s"),
    scratch_types=[
        pltpu.VMEM((K, C), jnp.int32),
        pltpu.VMEM((K, C), jnp.int32),
        pltpu.VMEM((C, D), jnp.float32),
        pltpu.VMEM((C, D), jnp.float32),
        pltpu.VMEM((ZR, D), jnp.float32),
        pltpu.VMEM_SHARED((NP, D), jnp.float32),
        pltpu.SemaphoreType.DMA,
        pltpu.SemaphoreType.DMA,
    ],
    compiler_params=pltpu.CompilerParams(use_tc_tiling_on_sc=False),
)(_sc_agg_body)


BN = 1000


def _tc_root_body(x_ref, wx_ref, b_ref, o_ref):
    dn = (((1,), (1,)), ((), ()))
    acc = lax.dot_general(x_ref[...], wx_ref[...], dn, preferred_element_type=jnp.float32)
    o_ref[...] = acc + b_ref[...]


def _tc_out_body(p_ref, root_ref, wr_ref, o_ref):
    agg = p_ref[0] + p_ref[1]
    dn = (((1,), (1,)), ((), ()))
    acc = lax.dot_general(agg, wr_ref[...], dn, preferred_element_type=jnp.float32)
    o_ref[...] = jnp.maximum(acc + root_ref[...], 0.0)


def kernel(x, edge_index, W_rel, b_rel, W_root):
    src = edge_index[0].reshape(NC, NS, K, C)
    dst = edge_index[1].reshape(NC, NS, K, C)
    part = _sc_agg(x, src, dst)

    root = pl.pallas_call(
        _tc_root_body,
        grid=(N // BN,),
        in_specs=[
            pl.BlockSpec((BN, D), lambda i: (i, 0)),
            pl.BlockSpec((D, D), lambda i: (0, 0)),
            pl.BlockSpec((1, D), lambda i: (0, 0)),
        ],
        out_specs=pl.BlockSpec((BN, D), lambda i: (i, 0)),
        out_shape=jax.ShapeDtypeStruct((N, D), jnp.float32),
    )(x, W_root, b_rel.reshape(1, D))

    out = pl.pallas_call(
        _tc_out_body,
        grid=(N // BN,),
        in_specs=[
            pl.BlockSpec((NC, BN, D), lambda i: (0, i, 0)),
            pl.BlockSpec((BN, D), lambda i: (i, 0)),
            pl.BlockSpec((D, D), lambda i: (0, 0)),
        ],
        out_specs=pl.BlockSpec((BN, D), lambda i: (i, 0)),
        out_shape=jax.ShapeDtypeStruct((N, D), jnp.float32),
    )(part, root, W_rel)
    return out


# speedup vs baseline: 1.0234x; 1.0178x over previous
import functools

import jax
import jax.numpy as jnp
from jax import lax
from jax.experimental import pallas as pl
from jax.experimental.pallas import tpu as pltpu
from jax.experimental.pallas import tpu_sc as plsc

N, E, D = 10000, 320000, 128
NC, NS = 2, 16
C = 100
K = E // (NC * NS * C)
NB = 2
assert E == NC * NS * K * C and K % NB == 0
NP = 10240
RPT = NP // NS
ZR = 8


def _sc_agg_body(x_hbm, src_hbm, dst_hbm, part_hbm,
                 sidx, didx, r0, r1, zbuf, agg_sh, s0, s1):
    cid = lax.axis_index("c")
    sid = lax.axis_index("s")
    rows = (r0, r1)
    sems = (s0, s1)

    pltpu.sync_copy(src_hbm.at[cid, sid], sidx)
    pltpu.sync_copy(dst_hbm.at[cid, sid], didx)

    for b in range(NB):
        pltpu.async_copy(x_hbm.at[sidx.at[b]], rows[b], sems[b])

    @pl.loop(0, ZR)
    def _zero_rows(r):
        @pl.loop(0, D // 16)
        def _zero_vecs(i):
            zbuf[r, pl.ds(i * 16, 16)] = jnp.zeros((16,), jnp.float32)

    base = sid * RPT

    @pl.loop(0, RPT // ZR)
    def _fill(z):
        pltpu.sync_copy(zbuf, agg_sh.at[pl.ds(base + z * ZR, ZR)])

    plsc.subcore_barrier()

    @pl.loop(0, K - NB, step=NB)
    def _edges(j):
        for b in range(NB):
            pltpu.make_async_copy(x_hbm.at[sidx.at[j + b]], rows[b], sems[b]).wait()
            pltpu.sync_copy(rows[b], agg_sh.at[didx.at[j + b]], add=True)
            pltpu.async_copy(x_hbm.at[sidx.at[j + NB + b]], rows[b], sems[b])

    for b in range(NB):
        pltpu.make_async_copy(x_hbm.at[sidx.at[K - NB + b]], rows[b], sems[b]).wait()
        pltpu.sync_copy(rows[b], agg_sh.at[didx.at[K - NB + b]], add=True)

    plsc.subcore_barrier()

    pltpu.sync_copy(agg_sh.at[pl.ds(base, RPT)],
                    part_hbm.at[cid, pl.ds(base, RPT)])


_sc_agg = functools.partial(
    pl.kernel,
    out_type=jax.ShapeDtypeStruct((NC, NP, D), jnp.float32),
    mesh=plsc.VectorSubcoreMesh(core_axis_name="c", subcore_axis_name="s"),
    scratch_types=[
        pltpu.VMEM((K, C), jnp.int32),
        pltpu.VMEM((K, C), jnp.int32),
        pltpu.VMEM((C, D), jnp.float32),
        pltpu.VMEM((C, D), jnp.float32),
        pltpu.VMEM((ZR, D), jnp.float32),
        pltpu.VMEM_SHARED((NP, D), jnp.float32),
        pltpu.SemaphoreType.DMA,
        pltpu.SemaphoreType.DMA,
    ],
    compiler_params=pltpu.CompilerParams(use_tc_tiling_on_sc=False),
)(_sc_agg_body)


BN = 2000


def _tc_body(p_ref, x_ref, wr_ref, wx_ref, b_ref, o_ref):
    agg = p_ref[0] + p_ref[1]
    dn = (((1,), (1,)), ((), ()))
    acc = lax.dot_general(agg, wr_ref[...], dn, preferred_element_type=jnp.float32)
    acc += lax.dot_general(x_ref[...], wx_ref[...], dn, preferred_element_type=jnp.float32)
    o_ref[...] = jnp.maximum(acc + b_ref[...], 0.0)


def kernel(x, edge_index, W_rel, b_rel, W_root):
    src = edge_index[0].reshape(NC, NS, K, C)
    dst = edge_index[1].reshape(NC, NS, K, C)
    part = _sc_agg(x, src, dst)

    out = pl.pallas_call(
        _tc_body,
        grid=(N // BN,),
        in_specs=[
            pl.BlockSpec((NC, BN, D), lambda i: (0, i, 0)),
            pl.BlockSpec((BN, D), lambda i: (i, 0)),
            pl.BlockSpec((D, D), lambda i: (0, 0)),
            pl.BlockSpec((D, D), lambda i: (0, 0)),
            pl.BlockSpec((1, D), lambda i: (0, 0)),
        ],
        out_specs=pl.BlockSpec((BN, D), lambda i: (i, 0)),
        out_shape=jax.ShapeDtypeStruct((N, D), jnp.float32),
    )(part, x, W_rel, W_root, b_rel.reshape(1, D))
    return out


# speedup vs baseline: 1.0350x; 1.0113x over previous
import functools

import jax
import jax.numpy as jnp
from jax import lax
from jax.experimental import pallas as pl
from jax.experimental.pallas import tpu as pltpu
from jax.experimental.pallas import tpu_sc as plsc

N, E, D = 10000, 320000, 128
NC, NS = 2, 16
C = 100
K = E // (NC * NS * C)
NB = 2
assert E == NC * NS * K * C and K % NB == 0
NP = 10240
RPT = NP // NS


def _sc_agg_body(x_hbm, src_hbm, dst_hbm, part_hbm,
                 sidx, didx, r0, r1, agg_sh, s0, s1):
    cid = lax.axis_index("c")
    sid = lax.axis_index("s")
    rows = (r0, r1)
    sems = (s0, s1)

    pltpu.sync_copy(src_hbm.at[cid, sid], sidx)
    pltpu.sync_copy(dst_hbm.at[cid, sid], didx)

    @pl.loop(0, C)
    def _zero_rows(r):
        @pl.loop(0, D // 16)
        def _zero_vecs(i):
            r0[r, pl.ds(i * 16, 16)] = jnp.zeros((16,), jnp.float32)

    base = sid * RPT

    @pl.loop(0, RPT // C)
    def _fill(z):
        pltpu.sync_copy(r0, agg_sh.at[pl.ds(base + z * C, C)])

    if RPT % C:
        pltpu.sync_copy(r0.at[pl.ds(0, RPT % C)],
                        agg_sh.at[pl.ds(base + (RPT // C) * C, RPT % C)])

    for b in range(NB):
        pltpu.async_copy(x_hbm.at[sidx.at[b]], rows[b], sems[b])

    plsc.subcore_barrier()

    @pl.loop(0, K - NB, step=NB)
    def _edges(j):
        for b in range(NB):
            pltpu.make_async_copy(x_hbm.at[sidx.at[j + b]], rows[b], sems[b]).wait()
            pltpu.sync_copy(rows[b], agg_sh.at[didx.at[j + b]], add=True)
            pltpu.async_copy(x_hbm.at[sidx.at[j + NB + b]], rows[b], sems[b])

    for b in range(NB):
        pltpu.make_async_copy(x_hbm.at[sidx.at[K - NB + b]], rows[b], sems[b]).wait()
        pltpu.sync_copy(rows[b], agg_sh.at[didx.at[K - NB + b]], add=True)

    plsc.subcore_barrier()

    pltpu.sync_copy(agg_sh.at[pl.ds(base, RPT)],
                    part_hbm.at[cid, pl.ds(base, RPT)])


_sc_agg = functools.partial(
    pl.kernel,
    out_type=jax.ShapeDtypeStruct((NC, NP, D), jnp.float32),
    mesh=plsc.VectorSubcoreMesh(core_axis_name="c", subcore_axis_name="s"),
    scratch_types=[
        pltpu.VMEM((K, C), jnp.int32),
        pltpu.VMEM((K, C), jnp.int32),
        pltpu.VMEM((C, D), jnp.float32),
        pltpu.VMEM((C, D), jnp.float32),
        pltpu.VMEM_SHARED((NP, D), jnp.float32),
        pltpu.SemaphoreType.DMA,
        pltpu.SemaphoreType.DMA,
    ],
    compiler_params=pltpu.CompilerParams(use_tc_tiling_on_sc=False),
)(_sc_agg_body)


BN = 2000


def _tc_body(p_ref, x_ref, wr_ref, wx_ref, b_ref, o_ref):
    agg = p_ref[0] + p_ref[1]
    dn = (((1,), (1,)), ((), ()))
    acc = lax.dot_general(agg, wr_ref[...], dn, preferred_element_type=jnp.float32)
    acc += lax.dot_general(x_ref[...], wx_ref[...], dn, preferred_element_type=jnp.float32)
    o_ref[...] = jnp.maximum(acc + b_ref[...], 0.0)


def kernel(x, edge_index, W_rel, b_rel, W_root):
    src = edge_index[0].reshape(NC, NS, K, C)
    dst = edge_index[1].reshape(NC, NS, K, C)
    part = _sc_agg(x, src, dst)

    out = pl.pallas_call(
        _tc_body,
        grid=(N // BN,),
        in_specs=[
            pl.BlockSpec((NC, BN, D), lambda i: (0, i, 0)),
            pl.BlockSpec((BN, D), lambda i: (i, 0)),
            pl.BlockSpec((D, D), lambda i: (0, 0)),
            pl.BlockSpec((D, D), lambda i: (0, 0)),
            pl.BlockSpec((1, D), lambda i: (0, 0)),
        ],
        out_specs=pl.BlockSpec((BN, D), lambda i: (i, 0)),
        out_shape=jax.ShapeDtypeStruct((N, D), jnp.float32),
    )(part, x, W_rel, W_root, b_rel.reshape(1, D))
    return out


# speedup vs baseline: 1.0582x; 1.0225x over previous
import functools

import jax
import jax.numpy as jnp
from jax import lax
from jax.experimental import pallas as pl
from jax.experimental.pallas import tpu as pltpu
from jax.experimental.pallas import tpu_sc as plsc

N, E, D = 10000, 320000, 128
NC, NS = 2, 16
C = 100
K = E // (NC * NS * C)
NB = 2
assert E == NC * NS * K * C and K % NB == 0
NP = 10240
RPT = NP // NS


def _sc_agg_body(xb_hbm, src_hbm, dst_hbm, zer_hbm, part_hbm,
                 sidx, didx, r0, r1, agg_sh, s0, s1):
    cid = lax.axis_index("c")
    sid = lax.axis_index("s")
    rows = (r0, r1)
    sems = (s0, s1)

    pltpu.sync_copy(src_hbm.at[cid, sid], sidx)
    pltpu.sync_copy(dst_hbm.at[cid, sid], didx)

    for b in range(NB):
        pltpu.async_copy(xb_hbm.at[sidx.at[b]], rows[b], sems[b])

    base = sid * RPT
    pltpu.sync_copy(zer_hbm, agg_sh.at[pl.ds(base, RPT)])

    plsc.subcore_barrier()

    @pl.loop(0, K - NB, step=NB)
    def _edges(j):
        for b in range(NB):
            pltpu.make_async_copy(xb_hbm.at[sidx.at[j + b]], rows[b], sems[b]).wait()
            pltpu.sync_copy(rows[b], agg_sh.at[didx.at[j + b]], add=True)
            pltpu.async_copy(xb_hbm.at[sidx.at[j + NB + b]], rows[b], sems[b])

    for b in range(NB):
        pltpu.make_async_copy(xb_hbm.at[sidx.at[K - NB + b]], rows[b], sems[b]).wait()
        pltpu.sync_copy(rows[b], agg_sh.at[didx.at[K - NB + b]], add=True)

    plsc.subcore_barrier()

    pltpu.sync_copy(agg_sh.at[pl.ds(base, RPT)],
                    part_hbm.at[cid, pl.ds(base, RPT)])


_sc_agg = functools.partial(
    pl.kernel,
    out_type=jax.ShapeDtypeStruct((NC, NP, D), jnp.bfloat16),
    mesh=plsc.VectorSubcoreMesh(core_axis_name="c", subcore_axis_name="s"),
    scratch_types=[
        pltpu.VMEM((K, C), jnp.int32),
        pltpu.VMEM((K, C), jnp.int32),
        pltpu.VMEM((C, D), jnp.bfloat16),
        pltpu.VMEM((C, D), jnp.bfloat16),
        pltpu.VMEM_SHARED((NP, D), jnp.bfloat16),
        pltpu.SemaphoreType.DMA,
        pltpu.SemaphoreType.DMA,
    ],
    compiler_params=pltpu.CompilerParams(use_tc_tiling_on_sc=False),
)(_sc_agg_body)


BN = 2000


def _tc_body(p_ref, x_ref, wr_ref, wx_ref, b_ref, o_ref):
    agg = (p_ref[0] + p_ref[1]).astype(jnp.float32)
    dn = (((1,), (1,)), ((), ()))
    acc = lax.dot_general(agg, wr_ref[...], dn, preferred_element_type=jnp.float32)
    acc += lax.dot_general(x_ref[...], wx_ref[...], dn, preferred_element_type=jnp.float32)
    o_ref[...] = jnp.maximum(acc + b_ref[...], 0.0)


def kernel(x, edge_index, W_rel, b_rel, W_root):
    xb = x.astype(jnp.bfloat16)
    src = edge_index[0].reshape(NC, NS, K, C)
    dst = edge_index[1].reshape(NC, NS, K, C)
    zer = jnp.zeros((RPT, D), jnp.bfloat16)
    part = _sc_agg(xb, src, dst, zer)

    out = pl.pallas_call(
        _tc_body,
        grid=(N // BN,),
        in_specs=[
            pl.BlockSpec((NC, BN, D), lambda i: (0, i, 0)),
            pl.BlockSpec((BN, D), lambda i: (i, 0)),
            pl.BlockSpec((D, D), lambda i: (0, 0)),
            pl.BlockSpec((D, D), lambda i: (0, 0)),
            pl.BlockSpec((1, D), lambda i: (0, 0)),
        ],
        out_specs=pl.BlockSpec((BN, D), lambda i: (i, 0)),
        out_shape=jax.ShapeDtypeStruct((N, D), jnp.float32),
    )(part, x, W_rel, W_root, b_rel.reshape(1, D))
    return out


# speedup vs baseline: 1.1524x; 1.0890x over previous
import functools

import jax
import jax.numpy as jnp
from jax import lax
from jax.experimental import pallas as pl
from jax.experimental.pallas import tpu as pltpu
from jax.experimental.pallas import tpu_sc as plsc

N, E, D = 10000, 320000, 128
NC, NS = 2, 16
C = 125
K = E // (NC * NS * C)
NB = 4
assert E == NC * NS * K * C and K % NB == 0
NP = 10240
RPT = NP // NS


def _sc_agg_body(xb_hbm, src_hbm, dst_hbm, zer_hbm, part_hbm,
                 sidx, didx, r0, r1, r2, r3, agg_sh,
                 s0, s1, s2, s3, t0, t1, t2, t3):
    cid = lax.axis_index("c")
    sid = lax.axis_index("s")
    rows = (r0, r1, r2, r3)
    sems = (s0, s1, s2, s3)
    scs = (t0, t1, t2, t3)

    pltpu.sync_copy(src_hbm.at[cid, sid], sidx)
    pltpu.sync_copy(dst_hbm.at[cid, sid], didx)

    for b in range(NB):
        pltpu.async_copy(xb_hbm.at[sidx.at[b]], rows[b], sems[b])

    base = sid * RPT
    pltpu.sync_copy(zer_hbm, agg_sh.at[pl.ds(base, RPT)])

    plsc.subcore_barrier()

    @pl.loop(0, K - NB, step=NB)
    def _edges(j):
        for b in range(NB):
            pltpu.make_async_copy(xb_hbm.at[sidx.at[j + b]], rows[b], sems[b]).wait()
            pltpu.async_copy(rows[b], agg_sh.at[didx.at[j + b]], scs[b], add=True)
        for b in range(NB):
            pltpu.make_async_copy(xb_hbm.at[sidx.at[j + b]], rows[b], scs[b]).wait()
            pltpu.async_copy(xb_hbm.at[sidx.at[j + NB + b]], rows[b], sems[b])

    for b in range(NB):
        pltpu.make_async_copy(xb_hbm.at[sidx.at[K - NB + b]], rows[b], sems[b]).wait()
        pltpu.async_copy(rows[b], agg_sh.at[didx.at[K - NB + b]], scs[b], add=True)
    for b in range(NB):
        pltpu.make_async_copy(xb_hbm.at[sidx.at[K - NB + b]], rows[b], scs[b]).wait()

    plsc.subcore_barrier()

    pltpu.sync_copy(agg_sh.at[pl.ds(base, RPT)],
                    part_hbm.at[cid, pl.ds(base, RPT)])


_sc_agg = functools.partial(
    pl.kernel,
    out_type=jax.ShapeDtypeStruct((NC, NP, D), jnp.bfloat16),
    mesh=plsc.VectorSubcoreMesh(core_axis_name="c", subcore_axis_name="s"),
    scratch_types=[
        pltpu.VMEM((K, C), jnp.int32),
        pltpu.VMEM((K, C), jnp.int32),
        pltpu.VMEM((C, D), jnp.bfloat16),
        pltpu.VMEM((C, D), jnp.bfloat16),
        pltpu.VMEM((C, D), jnp.bfloat16),
        pltpu.VMEM((C, D), jnp.bfloat16),
        pltpu.VMEM_SHARED((NP, D), jnp.bfloat16),
        pltpu.SemaphoreType.DMA,
        pltpu.SemaphoreType.DMA,
        pltpu.SemaphoreType.DMA,
        pltpu.SemaphoreType.DMA,
        pltpu.SemaphoreType.DMA,
        pltpu.SemaphoreType.DMA,
        pltpu.SemaphoreType.DMA,
        pltpu.SemaphoreType.DMA,
    ],
    compiler_params=pltpu.CompilerParams(use_tc_tiling_on_sc=False),
)(_sc_agg_body)


BN = 2000


def _tc_body(p_ref, x_ref, wr_ref, wx_ref, b_ref, o_ref):
    agg = (p_ref[0] + p_ref[1]).astype(jnp.float32)
    dn = (((1,), (1,)), ((), ()))
    acc = lax.dot_general(agg, wr_ref[...], dn, preferred_element_type=jnp.float32)
    acc += lax.dot_general(x_ref[...], wx_ref[...], dn, preferred_element_type=jnp.float32)
    o_ref[...] = jnp.maximum(acc + b_ref[...], 0.0)


def kernel(x, edge_index, W_rel, b_rel, W_root):
    xb = x.astype(jnp.bfloat16)
    src = edge_index[0].reshape(NC, NS, K, C)
    dst = edge_index[1].reshape(NC, NS, K, C)
    zer = jnp.zeros((RPT, D), jnp.bfloat16)
    part = _sc_agg(xb, src, dst, zer)

    out = pl.pallas_call(
        _tc_body,
        grid=(N // BN,),
        in_specs=[
            pl.BlockSpec((NC, BN, D), lambda i: (0, i, 0)),
            pl.BlockSpec((BN, D), lambda i: (i, 0)),
            pl.BlockSpec((D, D), lambda i: (0, 0)),
            pl.BlockSpec((D, D), lambda i: (0, 0)),
            pl.BlockSpec((1, D), lambda i: (0, 0)),
        ],
        out_specs=pl.BlockSpec((BN, D), lambda i: (i, 0)),
        out_shape=jax.ShapeDtypeStruct((N, D), jnp.float32),
    )(part, x, W_rel, W_root, b_rel.reshape(1, D))
    return out


# speedup vs baseline: 1.1887x; 1.0315x over previous
import functools

import jax
import jax.numpy as jnp
from jax import lax
from jax.experimental import pallas as pl
from jax.experimental.pallas import tpu as pltpu
from jax.experimental.pallas import tpu_sc as plsc

N, E, D = 10000, 320000, 128
NC, NS = 2, 16
C = 125
K = E // (NC * NS * C)
NB = 8
assert E == NC * NS * K * C and K % NB == 0
NP = 10240
RPT = NP // NS


def _sc_agg_body(xb_hbm, src_hbm, dst_hbm, zer_hbm, part_hbm,
                 sidx, didx, r0, r1, r2, r3, r4, r5, r6, r7, agg_sh,
                 s0, s1, s2, s3, s4, s5, s6, s7,
                 t0, t1, t2, t3, t4, t5, t6, t7):
    cid = lax.axis_index("c")
    sid = lax.axis_index("s")
    rows = (r0, r1, r2, r3, r4, r5, r6, r7)
    sems = (s0, s1, s2, s3, s4, s5, s6, s7)
    scs = (t0, t1, t2, t3, t4, t5, t6, t7)

    pltpu.sync_copy(src_hbm.at[cid, sid], sidx)
    pltpu.sync_copy(dst_hbm.at[cid, sid], didx)

    for b in range(NB):
        pltpu.async_copy(xb_hbm.at[sidx.at[b]], rows[b], sems[b])

    base = sid * RPT
    pltpu.sync_copy(zer_hbm, agg_sh.at[pl.ds(base, RPT)])

    plsc.subcore_barrier()

    @pl.loop(0, K - NB, step=NB)
    def _edges(j):
        for b in range(NB):
            pltpu.make_async_copy(xb_hbm.at[sidx.at[j + b]], rows[b], sems[b]).wait()
            pltpu.async_copy(rows[b], agg_sh.at[didx.at[j + b]], scs[b], add=True)
        for b in range(NB):
            pltpu.make_async_copy(xb_hbm.at[sidx.at[j + b]], rows[b], scs[b]).wait()
            pltpu.async_copy(xb_hbm.at[sidx.at[j + NB + b]], rows[b], sems[b])

    for b in range(NB):
        pltpu.make_async_copy(xb_hbm.at[sidx.at[K - NB + b]], rows[b], sems[b]).wait()
        pltpu.async_copy(rows[b], agg_sh.at[didx.at[K - NB + b]], scs[b], add=True)
    for b in range(NB):
        pltpu.make_async_copy(xb_hbm.at[sidx.at[K - NB + b]], rows[b], scs[b]).wait()

    plsc.subcore_barrier()

    pltpu.sync_copy(agg_sh.at[pl.ds(base, RPT)],
                    part_hbm.at[cid, pl.ds(base, RPT)])


_sc_agg = functools.partial(
    pl.kernel,
    out_type=jax.ShapeDtypeStruct((NC, NP, D), jnp.bfloat16),
    mesh=plsc.VectorSubcoreMesh(core_axis_name="c", subcore_axis_name="s"),
    scratch_types=[
        pltpu.VMEM((K, C), jnp.int32),
        pltpu.VMEM((K, C), jnp.int32),
        pltpu.VMEM((C, D), jnp.bfloat16),
        pltpu.VMEM((C, D), jnp.bfloat16),
        pltpu.VMEM((C, D), jnp.bfloat16),
        pltpu.VMEM((C, D), jnp.bfloat16),
        pltpu.VMEM((C, D), jnp.bfloat16),
        pltpu.VMEM((C, D), jnp.bfloat16),
        pltpu.VMEM((C, D), jnp.bfloat16),
        pltpu.VMEM((C, D), jnp.bfloat16),
        pltpu.VMEM_SHARED((NP, D), jnp.bfloat16),
        pltpu.SemaphoreType.DMA,
        pltpu.SemaphoreType.DMA,
        pltpu.SemaphoreType.DMA,
        pltpu.SemaphoreType.DMA,
        pltpu.SemaphoreType.DMA,
        pltpu.SemaphoreType.DMA,
        pltpu.SemaphoreType.DMA,
        pltpu.SemaphoreType.DMA,
        pltpu.SemaphoreType.DMA,
        pltpu.SemaphoreType.DMA,
        pltpu.SemaphoreType.DMA,
        pltpu.SemaphoreType.DMA,
        pltpu.SemaphoreType.DMA,
        pltpu.SemaphoreType.DMA,
        pltpu.SemaphoreType.DMA,
        pltpu.SemaphoreType.DMA,
    ],
    compiler_params=pltpu.CompilerParams(use_tc_tiling_on_sc=False),
)(_sc_agg_body)


BN = 2000


def _tc_body(p_ref, x_ref, wr_ref, wx_ref, b_ref, o_ref):
    agg = (p_ref[0] + p_ref[1]).astype(jnp.float32)
    dn = (((1,), (1,)), ((), ()))
    acc = lax.dot_general(agg, wr_ref[...], dn, preferred_element_type=jnp.float32)
    acc += lax.dot_general(x_ref[...], wx_ref[...], dn, preferred_element_type=jnp.float32)
    o_ref[...] = jnp.maximum(acc + b_ref[...], 0.0)


def kernel(x, edge_index, W_rel, b_rel, W_root):
    xb = x.astype(jnp.bfloat16)
    src = edge_index[0].reshape(NC, NS, K, C)
    dst = edge_index[1].reshape(NC, NS, K, C)
    zer = jnp.zeros((RPT, D), jnp.bfloat16)
    part = _sc_agg(xb, src, dst, zer)

    out = pl.pallas_call(
        _tc_body,
        grid=(N // BN,),
        in_specs=[
            pl.BlockSpec((NC, BN, D), lambda i: (0, i, 0)),
            pl.BlockSpec((BN, D), lambda i: (i, 0)),
            pl.BlockSpec((D, D), lambda i: (0, 0)),
            pl.BlockSpec((D, D), lambda i: (0, 0)),
            pl.BlockSpec((1, D), lambda i: (0, 0)),
        ],
        out_specs=pl.BlockSpec((BN, D), lambda i: (i, 0)),
        out_shape=jax.ShapeDtypeStruct((N, D), jnp.float32),
    )(part, x, W_rel, W_root, b_rel.reshape(1, D))
    return out
